# Initial kernel scaffold; baseline (speedup 1.0000x reference)
import jax
import jax.numpy as jnp
from jax.experimental import pallas as pl


def kernel(x, embed):
    raise NotImplementedError("write your pallas kernel here")


# speedup vs baseline: 1.9280x; 1.9280x over previous
import jax
import jax.numpy as jnp
from jax.experimental import pallas as pl

DIM = 64
CODEBOOK_SIZE = 1024
B = 16
N = 1024
BN = B * N
TILE = 1024


def _vq_kernel(x_ref, x2_ref, e_ref, e2_ref, dist_ref, q_ref, ind_ref):
    x = x_ref[...]
    x2 = x2_ref[...]
    e = e_ref[...]
    e2 = e2_ref[...]
    xy = jax.lax.dot_general(
        x, e, (((1,), (1,)), ((), ())),
        preferred_element_type=jnp.float32,
    ) * -2.0
    d2 = (x2 + e2) + xy
    s = jnp.sqrt(jnp.maximum(d2, 0.0))
    dist_ref[...] = -s
    mins = jnp.min(s, axis=1, keepdims=True)
    iota = jax.lax.broadcasted_iota(jnp.int32, (TILE, CODEBOOK_SIZE), 1)
    ind = jnp.min(jnp.where(s <= mins, iota, CODEBOOK_SIZE), axis=1,
                  keepdims=True)
    ind_ref[...] = ind
    oh = jnp.where(iota == ind, 1.0, 0.0)
    q_ref[...] = jax.lax.dot_general(
        oh, e, (((1,), (0,)), ((), ())),
        precision=jax.lax.Precision.HIGHEST,
        preferred_element_type=jnp.float32,
    )


def kernel(x, embed):
    e = embed[0]
    e2 = jnp.sum(e ** 2, axis=-1)[None, :]
    xf = x.reshape(BN, DIM)
    x2 = jnp.sum(xf ** 2, axis=-1)[:, None]
    grid = (BN // TILE,)
    dist, q, ind = pl.pallas_call(
        _vq_kernel,
        grid=grid,
        in_specs=[
            pl.BlockSpec((TILE, DIM), lambda i: (i, 0)),
            pl.BlockSpec((TILE, 1), lambda i: (i, 0)),
            pl.BlockSpec((CODEBOOK_SIZE, DIM), lambda i: (0, 0)),
            pl.BlockSpec((1, CODEBOOK_SIZE), lambda i: (0, 0)),
        ],
        out_specs=[
            pl.BlockSpec((TILE, CODEBOOK_SIZE), lambda i: (i, 0)),
            pl.BlockSpec((TILE, DIM), lambda i: (i, 0)),
            pl.BlockSpec((TILE, 1), lambda i: (i, 0)),
        ],
        out_shape=[
            jax.ShapeDtypeStruct((BN, CODEBOOK_SIZE), jnp.float32),
            jax.ShapeDtypeStruct((BN, DIM), jnp.float32),
            jax.ShapeDtypeStruct((BN, 1), jnp.int32),
        ],
    )(xf, x2, e, e2)
    quantize = q.reshape(B, N, DIM)
    embed_ind = ind.reshape(B, N)
    dist_out = dist.reshape(1, B, N, CODEBOOK_SIZE)
    return (quantize, embed_ind, dist_out)


# speedup vs baseline: 2.9404x; 1.5251x over previous
import jax
import jax.numpy as jnp
from jax.experimental import pallas as pl
from jax.experimental.pallas import tpu as pltpu

DIM = 64
CODEBOOK_SIZE = 1024
B = 16
N = 1024
BN = B * N
TILE = 1024


def _vq_kernel(x_ref, x2_ref, e_ref, e2_ref, dist_ref, q_ref, ind_ref):
    x = x_ref[...]
    x2 = x2_ref[...]
    e = e_ref[...]
    e2 = e2_ref[...]
    xy = jax.lax.dot_general(
        x, e, (((1,), (1,)), ((), ())),
        preferred_element_type=jnp.float32,
    ) * -2.0
    d2 = (x2 + e2) + xy
    s = jnp.sqrt(jnp.maximum(d2, 0.0))
    dist_ref[...] = -s
    mins = jnp.min(s, axis=1, keepdims=True)
    iota = jax.lax.broadcasted_iota(jnp.int32, (TILE, CODEBOOK_SIZE), 1)
    ind = jnp.min(jnp.where(s <= mins, iota, CODEBOOK_SIZE), axis=1,
                  keepdims=True)
    ind_ref[...] = ind
    oh = jnp.where(iota == ind, 1.0, 0.0)
    q_ref[...] = jax.lax.dot_general(
        oh, e, (((1,), (0,)), ((), ())),
        preferred_element_type=jnp.float32,
    )


def kernel(x, embed):
    e = embed[0]
    e2 = jnp.sum(e ** 2, axis=-1)[None, :]
    xf = x.reshape(BN, DIM)
    x2 = jnp.sum(xf ** 2, axis=-1)[:, None]
    grid = (BN // TILE,)
    dist, q, ind = pl.pallas_call(
        _vq_kernel,
        grid=grid,
        in_specs=[
            pl.BlockSpec((TILE, DIM), lambda i: (i, 0)),
            pl.BlockSpec((TILE, 1), lambda i: (i, 0)),
            pl.BlockSpec((CODEBOOK_SIZE, DIM), lambda i: (0, 0)),
            pl.BlockSpec((1, CODEBOOK_SIZE), lambda i: (0, 0)),
        ],
        out_specs=[
            pl.BlockSpec((TILE, CODEBOOK_SIZE), lambda i: (i, 0)),
            pl.BlockSpec((TILE, DIM), lambda i: (i, 0)),
            pl.BlockSpec((TILE, 1), lambda i: (i, 0)),
        ],
        out_shape=[
            jax.ShapeDtypeStruct((BN, CODEBOOK_SIZE), jnp.float32),
            jax.ShapeDtypeStruct((BN, DIM), jnp.float32),
            jax.ShapeDtypeStruct((BN, 1), jnp.int32),
        ],
        compiler_params=pltpu.CompilerParams(
            dimension_semantics=("parallel",)),
    )(xf, x2, e, e2)
    quantize = q.reshape(B, N, DIM)
    embed_ind = ind.reshape(B, N)
    dist_out = dist.reshape(1, B, N, CODEBOOK_SIZE)
    return (quantize, embed_ind, dist_out)
